# baseline (speedup 1.0000x reference)
import functools

import jax
import jax.numpy as jnp
from jax import lax
from jax.experimental import pallas as pl
from jax.experimental.pallas import tpu as pltpu
from jax.experimental.pallas import tpu_sc as plsc

N_NODES = 10000
N_EDGES = 320000
D = 128
D_EDGE = 16

NC = 2
NS = 16
NW = NC * NS
EDGES_PER_TILE = N_EDGES // NW
CHUNK = 40
SUPER = 25
NSTEPS = EDGES_PER_TILE // CHUNK
NSUPER = NSTEPS // SUPER
WB_TILES = 10
WB_ROWS = N_NODES // WB_TILES


def _proj_body(x_ref, w_ref, ps_ref, pd_ref):
    xa = x_ref[...]
    ps_ref[...] = jnp.dot(xa, w_ref[0:D, :], preferred_element_type=jnp.float32)
    pd_ref[...] = jnp.dot(xa, w_ref[D:2 * D, :], preferred_element_type=jnp.float32)


def _node_proj(x, w1ab):
    return pl.pallas_call(
        _proj_body,
        grid=(10,),
        in_specs=[
            pl.BlockSpec((1000, D), lambda i: (i, 0)),
            pl.BlockSpec((2 * D, D), lambda i: (0, 0)),
        ],
        out_specs=[
            pl.BlockSpec((1000, D), lambda i: (i, 0)),
            pl.BlockSpec((1000, D), lambda i: (i, 0)),
        ],
        out_shape=[
            jax.ShapeDtypeStruct((N_NODES, D), jnp.float32),
            jax.ShapeDtypeStruct((N_NODES, D), jnp.float32),
        ],
    )(x, w1ab)


def _edge_proj_body(ea_ref, w_ref, b_ref, e_ref):
    e_ref[...] = (
        jnp.dot(ea_ref[...].astype(jnp.bfloat16),
                w_ref[...].astype(jnp.bfloat16),
                preferred_element_type=jnp.float32)
        + b_ref[...]
    )


def _edge_proj(edge_attr, w1c, b1):
    return pl.pallas_call(
        _edge_proj_body,
        grid=(40,),
        in_specs=[
            pl.BlockSpec((8000, D_EDGE), lambda i: (i, 0)),
            pl.BlockSpec((D_EDGE, D), lambda i: (0, 0)),
            pl.BlockSpec((1, D), lambda i: (0, 0)),
        ],
        out_specs=pl.BlockSpec((8000, D), lambda i: (i, 0)),
        out_shape=jax.ShapeDtypeStruct((N_EDGES, D), jnp.float32),
    )(edge_attr, w1c, b1.reshape(1, D))


def _edge_agg_body(ps_hbm, pd_hbm, e_hbm, idx_hbm, out_hbm,
                   idx_v, ps0, pd0, e0, ps1, pd1, e1, agg_sh,
                   sg0, sg1, ss0, ss1):
    c = lax.axis_index("c")
    s = lax.axis_index("s")
    t = c * NS + s
    base_t = t * EDGES_PER_TILE

    @pl.when(s < WB_TILES)
    def _zero():
        def _zrow(r, _):
            for j in range(D // 16):
                e0[r, pl.ds(j * 16, 16)] = jnp.zeros((16,), jnp.float32)
            return 0
        lax.fori_loop(0, CHUNK, _zrow, 0)
        for k in range(WB_ROWS // CHUNK):
            pltpu.sync_copy(e0, agg_sh.at[pl.ds(s * WB_ROWS + k * CHUNK, CHUNK)])
    plsc.subcore_barrier()

    def issue_gathers(i, psb, pdb, eb, sem):
        slot = lax.rem(i // SUPER, 2)
        row = lax.rem(i, SUPER)
        pltpu.async_copy(ps_hbm.at[idx_v.at[slot, 2 * row]], psb, sem)
        pltpu.async_copy(pd_hbm.at[idx_v.at[slot, 2 * row + 1]], pdb, sem)
        pltpu.async_copy(e_hbm.at[pl.ds(base_t + i * CHUNK, CHUNK)], eb, sem)

    def wait_gathers(psb, pdb, eb, sem):
        pltpu.make_async_copy(ps_hbm.at[pl.ds(0, CHUNK)], psb, sem).wait()
        pltpu.make_async_copy(pd_hbm.at[pl.ds(0, CHUNK)], pdb, sem).wait()
        pltpu.make_async_copy(e_hbm.at[pl.ds(0, CHUNK)], eb, sem).wait()

    def wait_scat(psb, sem):
        pltpu.make_async_copy(e_hbm.at[pl.ds(0, CHUNK)], psb, sem).wait()

    pltpu.sync_copy(idx_hbm.at[t, 0], idx_v.at[0])
    issue_gathers(0, ps0, pd0, e0, sg0)

    ZERO = jnp.zeros((16,), jnp.float32)

    def step(i, psb, pdb, eb, sgp, ssp, psq, pdq, eq, sgq, ssq):
        @pl.when(i >= 1)
        def _wscat():
            wait_scat(psq, ssq)

        @pl.when(i + 1 < NSTEPS)
        def _prefetch():
            k1 = (i + 1) // SUPER
            @pl.when(lax.rem(i + 1, SUPER) == 0)
            def _superchunk():
                pltpu.sync_copy(idx_hbm.at[t, k1], idx_v.at[lax.rem(k1, 2)])
            issue_gathers(i + 1, psq, pdq, eq, sgq)

        wait_gathers(psb, pdb, eb, sgp)

        def _row(r2, _):
            for sub in range(2):
                r = 2 * r2 + sub
                for j in range(D // 16):
                    sl = pl.ds(j * 16, 16)
                    v = psb[r, sl] + pdb[r, sl] + eb[r, sl]
                    psb[r, sl] = jnp.maximum(v, ZERO)
            return 0
        lax.fori_loop(0, CHUNK // 2, _row, 0)

        slot = lax.rem(i // SUPER, 2)
        row = lax.rem(i, SUPER)
        pltpu.async_copy(psb, agg_sh.at[idx_v.at[slot, 2 * row + 1]], ssp, add=True)

    def _two_steps(j, _):
        i = 2 * j
        step(i, ps0, pd0, e0, sg0, ss0, ps1, pd1, e1, sg1, ss1)
        step(i + 1, ps1, pd1, e1, sg1, ss1, ps0, pd0, e0, sg0, ss0)
        return 0

    lax.fori_loop(0, NSTEPS // 2, _two_steps, 0)
    wait_scat(ps1, ss1)
    plsc.subcore_barrier()

    @pl.when(s < WB_TILES)
    def _writeback():
        pltpu.sync_copy(agg_sh.at[pl.ds(s * WB_ROWS, WB_ROWS)],
                        out_hbm.at[c, pl.ds(s * WB_ROWS, WB_ROWS)])


def _edge_agg(ps, pd, e, idx_packed):
    mesh = plsc.VectorSubcoreMesh(core_axis_name="c", subcore_axis_name="s")
    k = functools.partial(
        pl.kernel,
        out_type=jax.ShapeDtypeStruct((NC, N_NODES, D), jnp.float32),
        mesh=mesh,
        scratch_types=[
            pltpu.VMEM((2, 2 * SUPER, CHUNK), jnp.int32),
            pltpu.VMEM((CHUNK, D), jnp.float32),
            pltpu.VMEM((CHUNK, D), jnp.float32),
            pltpu.VMEM((CHUNK, D), jnp.float32),
            pltpu.VMEM((CHUNK, D), jnp.float32),
            pltpu.VMEM((CHUNK, D), jnp.float32),
            pltpu.VMEM((CHUNK, D), jnp.float32),
            pltpu.VMEM_SHARED((N_NODES, D), jnp.float32),
            pltpu.SemaphoreType.DMA,
            pltpu.SemaphoreType.DMA,
            pltpu.SemaphoreType.DMA,
            pltpu.SemaphoreType.DMA,
        ],
    )(_edge_agg_body)
    return k(ps, pd, e, idx_packed)


def _gru_body(part_ref, x_ref, wx_ref, wh_ref, b_ref, out_ref):
    agg = part_ref[0] + part_ref[1]
    xa = x_ref[...]
    gx = jnp.dot(agg, wx_ref[...], preferred_element_type=jnp.float32) + b_ref[...]
    gh = jnp.dot(xa, wh_ref[...], preferred_element_type=jnp.float32)
    z = jax.nn.sigmoid(gx[:, 0:D] + gh[:, 0:D])
    r = jax.nn.sigmoid(gx[:, D:2 * D] + gh[:, D:2 * D])
    h_cand = jnp.tanh(gx[:, 2 * D:3 * D] + r * gh[:, 2 * D:3 * D])
    out_ref[...] = (1.0 - z) * xa + z * h_cand


def _gru(part, x, wx, wh, b):
    return pl.pallas_call(
        _gru_body,
        grid=(10,),
        in_specs=[
            pl.BlockSpec((NC, 1000, D), lambda i: (0, i, 0)),
            pl.BlockSpec((1000, D), lambda i: (i, 0)),
            pl.BlockSpec((D, 3 * D), lambda i: (0, 0)),
            pl.BlockSpec((D, 3 * D), lambda i: (0, 0)),
            pl.BlockSpec((1, 3 * D), lambda i: (0, 0)),
        ],
        out_specs=pl.BlockSpec((1000, D), lambda i: (i, 0)),
        out_shape=jax.ShapeDtypeStruct((N_NODES, D), jnp.float32),
    )(part, x, wx, wh, b.reshape(1, 3 * D))


def kernel(x, edge_index, edge_attr, W1, b1, Wx, Wh, b):
    src = edge_index[0].astype(jnp.int32).reshape(NW, NSUPER, SUPER, CHUNK)
    dst = edge_index[1].astype(jnp.int32).reshape(NW, NSUPER, SUPER, CHUNK)
    idx_packed = jnp.stack([src, dst], axis=3).reshape(NW, NSUPER, 2 * SUPER, CHUNK)
    ps, pd = _node_proj(x, W1[:2 * D])
    e = _edge_proj(edge_attr, W1[2 * D:], b1)
    part = _edge_agg(ps, pd, e, idx_packed)
    return _gru(part, x, Wx, Wh, b)


# speedup vs baseline: 1.2897x; 1.2897x over previous
import functools

import jax
import jax.numpy as jnp
from jax import lax
from jax.experimental import pallas as pl
from jax.experimental.pallas import tpu as pltpu
from jax.experimental.pallas import tpu_sc as plsc

N_NODES = 10000
N_EDGES = 320000
D = 128
D_EDGE = 16

NC = 2
NS = 16
NW = NC * NS
EDGES_PER_TILE = N_EDGES // NW
CHUNK = 40
SUPER = 25
NSTEPS = EDGES_PER_TILE // CHUNK
NSUPER = NSTEPS // SUPER
WB_TILES = 10
WB_ROWS = N_NODES // WB_TILES


def _proj_body(x_ref, w_ref, ps_ref, pd_ref):
    xa = x_ref[...]
    ps_ref[...] = jnp.dot(xa, w_ref[0:D, :], preferred_element_type=jnp.float32)
    pd_ref[...] = jnp.dot(xa, w_ref[D:2 * D, :], preferred_element_type=jnp.float32)


def _node_proj(x, w1ab):
    return pl.pallas_call(
        _proj_body,
        grid=(10,),
        in_specs=[
            pl.BlockSpec((1000, D), lambda i: (i, 0)),
            pl.BlockSpec((2 * D, D), lambda i: (0, 0)),
        ],
        out_specs=[
            pl.BlockSpec((1000, D), lambda i: (i, 0)),
            pl.BlockSpec((1000, D), lambda i: (i, 0)),
        ],
        out_shape=[
            jax.ShapeDtypeStruct((N_NODES, D), jnp.float32),
            jax.ShapeDtypeStruct((N_NODES, D), jnp.float32),
        ],
    )(x, w1ab)


def _edge_proj_body(eat_ref, w_ref, b_ref, e_ref):
    e_ref[...] = (
        lax.dot_general(eat_ref[...], w_ref[...],
                        (((0,), (0,)), ((), ())),
                        preferred_element_type=jnp.float32)
        + b_ref[...]
    )


def _edge_proj(edge_attr_t, w1c, b1):
    return pl.pallas_call(
        _edge_proj_body,
        grid=(50,),
        in_specs=[
            pl.BlockSpec((D_EDGE, 6400), lambda i: (0, i)),
            pl.BlockSpec((D_EDGE, D), lambda i: (0, 0)),
            pl.BlockSpec((1, D), lambda i: (0, 0)),
        ],
        out_specs=pl.BlockSpec((6400, D), lambda i: (i, 0)),
        out_shape=jax.ShapeDtypeStruct((N_EDGES, D), jnp.float32),
    )(edge_attr_t, w1c, b1.reshape(1, D))


def _edge_agg_body(ps_hbm, pd_hbm, e_hbm, src_hbm, dst_hbm, out_hbm,
                   sidx_v, didx_v, ps0, pd0, e0, ps1, pd1, e1, agg_sh,
                   sg0, sg1, ss0, ss1):
    c = lax.axis_index("c")
    s = lax.axis_index("s")
    t = c * NS + s
    base_t = t * EDGES_PER_TILE

    @pl.when(s < WB_TILES)
    def _zero():
        def _zrow(r, _):
            for j in range(D // 16):
                e0[r, pl.ds(j * 16, 16)] = jnp.zeros((16,), jnp.float32)
            return 0
        lax.fori_loop(0, CHUNK, _zrow, 0)
        for k in range(WB_ROWS // CHUNK):
            pltpu.sync_copy(e0, agg_sh.at[pl.ds(s * WB_ROWS + k * CHUNK, CHUNK)])
    plsc.subcore_barrier()

    def issue_gathers(i, psb, pdb, eb, sem):
        slot = lax.rem(i // SUPER, 2)
        row = lax.rem(i, SUPER)
        pltpu.async_copy(ps_hbm.at[sidx_v.at[slot, row]], psb, sem)
        pltpu.async_copy(pd_hbm.at[didx_v.at[slot, row]], pdb, sem)
        pltpu.async_copy(e_hbm.at[pl.ds(base_t + i * CHUNK, CHUNK)], eb, sem)

    def wait_gathers(psb, pdb, eb, sem):
        pltpu.make_async_copy(ps_hbm.at[pl.ds(0, CHUNK)], psb, sem).wait()
        pltpu.make_async_copy(pd_hbm.at[pl.ds(0, CHUNK)], pdb, sem).wait()
        pltpu.make_async_copy(e_hbm.at[pl.ds(0, CHUNK)], eb, sem).wait()

    def wait_scat(psb, sem):
        pltpu.make_async_copy(e_hbm.at[pl.ds(0, CHUNK)], psb, sem).wait()

    pltpu.sync_copy(src_hbm.at[t, 0], sidx_v.at[0])
    pltpu.sync_copy(dst_hbm.at[t, 0], didx_v.at[0])
    issue_gathers(0, ps0, pd0, e0, sg0)

    ZERO = jnp.zeros((16,), jnp.float32)

    def step(i, psb, pdb, eb, sgp, ssp, psq, pdq, eq, sgq, ssq):
        @pl.when(i >= 1)
        def _wscat():
            wait_scat(psq, ssq)

        @pl.when(i + 1 < NSTEPS)
        def _prefetch():
            k1 = (i + 1) // SUPER
            @pl.when(lax.rem(i + 1, SUPER) == 0)
            def _superchunk():
                slot1 = lax.rem(k1, 2)
                pltpu.sync_copy(src_hbm.at[t, k1], sidx_v.at[slot1])
                pltpu.sync_copy(dst_hbm.at[t, k1], didx_v.at[slot1])
            issue_gathers(i + 1, psq, pdq, eq, sgq)

        wait_gathers(psb, pdb, eb, sgp)

        def _row(r2, _):
            for sub in range(2):
                r = 2 * r2 + sub
                for j in range(D // 16):
                    sl = pl.ds(j * 16, 16)
                    v = psb[r, sl] + pdb[r, sl] + eb[r, sl]
                    psb[r, sl] = jnp.maximum(v, ZERO)
            return 0
        lax.fori_loop(0, CHUNK // 2, _row, 0)

        slot = lax.rem(i // SUPER, 2)
        row = lax.rem(i, SUPER)
        pltpu.async_copy(psb, agg_sh.at[didx_v.at[slot, row]], ssp, add=True)

    def _two_steps(j, _):
        i = 2 * j
        step(i, ps0, pd0, e0, sg0, ss0, ps1, pd1, e1, sg1, ss1)
        step(i + 1, ps1, pd1, e1, sg1, ss1, ps0, pd0, e0, sg0, ss0)
        return 0

    lax.fori_loop(0, NSTEPS // 2, _two_steps, 0)
    wait_scat(ps1, ss1)
    plsc.subcore_barrier()

    @pl.when(s < WB_TILES)
    def _writeback():
        pltpu.sync_copy(agg_sh.at[pl.ds(s * WB_ROWS, WB_ROWS)],
                        out_hbm.at[c, pl.ds(s * WB_ROWS, WB_ROWS)])


def _edge_agg(ps, pd, e, src_flat, dst4):
    mesh = plsc.VectorSubcoreMesh(core_axis_name="c", subcore_axis_name="s")
    k = functools.partial(
        pl.kernel,
        out_type=jax.ShapeDtypeStruct((NC, N_NODES, D), jnp.float32),
        mesh=mesh,
        scratch_types=[
            pltpu.VMEM((2, SUPER, CHUNK), jnp.int32),
            pltpu.VMEM((2, SUPER, CHUNK), jnp.int32),
            pltpu.VMEM((CHUNK, D), jnp.float32),
            pltpu.VMEM((CHUNK, D), jnp.float32),
            pltpu.VMEM((CHUNK, D), jnp.float32),
            pltpu.VMEM((CHUNK, D), jnp.float32),
            pltpu.VMEM((CHUNK, D), jnp.float32),
            pltpu.VMEM((CHUNK, D), jnp.float32),
            pltpu.VMEM_SHARED((N_NODES, D), jnp.float32),
            pltpu.SemaphoreType.DMA,
            pltpu.SemaphoreType.DMA,
            pltpu.SemaphoreType.DMA,
            pltpu.SemaphoreType.DMA,
        ],
    )(_edge_agg_body)
    return k(ps, pd, e, src_flat, dst4)


def _gru_body(part_ref, x_ref, wx_ref, wh_ref, b_ref, out_ref):
    agg = part_ref[0] + part_ref[1]
    xa = x_ref[...]
    gx = jnp.dot(agg, wx_ref[...], preferred_element_type=jnp.float32) + b_ref[...]
    gh = jnp.dot(xa, wh_ref[...], preferred_element_type=jnp.float32)
    z = jax.nn.sigmoid(gx[:, 0:D] + gh[:, 0:D])
    r = jax.nn.sigmoid(gx[:, D:2 * D] + gh[:, D:2 * D])
    h_cand = jnp.tanh(gx[:, 2 * D:3 * D] + r * gh[:, 2 * D:3 * D])
    out_ref[...] = (1.0 - z) * xa + z * h_cand


def _gru(part, x, wx, wh, b):
    return pl.pallas_call(
        _gru_body,
        grid=(10,),
        in_specs=[
            pl.BlockSpec((NC, 1000, D), lambda i: (0, i, 0)),
            pl.BlockSpec((1000, D), lambda i: (i, 0)),
            pl.BlockSpec((D, 3 * D), lambda i: (0, 0)),
            pl.BlockSpec((D, 3 * D), lambda i: (0, 0)),
            pl.BlockSpec((1, 3 * D), lambda i: (0, 0)),
        ],
        out_specs=pl.BlockSpec((1000, D), lambda i: (i, 0)),
        out_shape=jax.ShapeDtypeStruct((N_NODES, D), jnp.float32),
    )(part, x, wx, wh, b.reshape(1, 3 * D))


def kernel(x, edge_index, edge_attr, W1, b1, Wx, Wh, b):
    src_flat = edge_index[0].astype(jnp.int32).reshape(NW, NSUPER, SUPER, CHUNK)
    dst4 = edge_index[1].astype(jnp.int32).reshape(NW, NSUPER, SUPER, CHUNK)
    ps, pd = _node_proj(x, W1[:2 * D])
    e = _edge_proj(edge_attr.T, W1[2 * D:], b1)
    part = _edge_agg(ps, pd, e, src_flat, dst4)
    return _gru(part, x, Wx, Wh, b)
